# baseline (speedup 1.0000x reference)
import functools

import jax
import jax.numpy as jnp
from jax import lax
from jax.experimental import pallas as pl
from jax.experimental.pallas import tpu as pltpu
from jax.experimental.pallas import tpu_sc as plsc

_K = 64


@functools.cache
def _make_sc_segsum(e, R, W_pad, NB):
    assert R % 16 == 0 and W_pad % 32 == 0
    mesh = plsc.VectorSubcoreMesh(
        core_axis_name="c", subcore_axis_name="s", num_cores=2, num_subcores=16
    )

    def body(D, relblk, meta, h, accn, accd, dbuf, relbuf, meta_v):
        c = lax.axis_index("c")
        t = lax.axis_index("s")
        wid = c * 16 + t
        pltpu.sync_copy(meta, meta_v)

        def window(wi, carry):
            w = wi * 32 + wid

            def zr(r, carry2):
                z = jnp.zeros((16,), jnp.float32)
                for f in range(16):
                    accn[r, pl.ds(f * 16, 16)] = z
                    accd[r, pl.ds(f * 16, 16)] = z
                return carry2

            lax.fori_loop(0, R, zr, 0)

            nb = meta_v[pl.ds(w, 16)][0]
            ts_ = meta_v[pl.ds(W_pad + w, 16)][0]
            bb = meta_v[pl.ds(2 * W_pad + w, 16)][0]

            def blk(j, carry2):
                s = pl.multiple_of(ts_ + j * _K, 8)
                pltpu.sync_copy(D.at[pl.ds(s, _K)], dbuf)
                pltpu.sync_copy(
                    relblk.at[pl.ds((bb + j) * (2 * _K), _K)],
                    relbuf.at[pl.ds(0, _K)],
                )

                def edge(i, carry3):
                    rel = relbuf[pl.ds(i, 16)][0]
                    for f in range(16):
                        sl = pl.ds(f * 16, 16)
                        accn[rel, sl] = accn[rel, sl] + dbuf[i, sl]
                        accd[rel, sl] = accd[rel, sl] + dbuf[i, pl.ds(256 + f * 16, 16)]
                    return carry3

                lax.fori_loop(0, _K, edge, 0)
                return carry2

            lax.fori_loop(0, nb, blk, 0)

            def dv(r, carry2):
                for f in range(16):
                    sl = pl.ds(f * 16, 16)
                    accn[r, sl] = accn[r, sl] / (accd[r, sl] + 1e-6)
                return carry2

            lax.fori_loop(0, R, dv, 0)
            pltpu.sync_copy(
                accn.at[pl.ds(0, R)], h.at[pl.ds(pl.multiple_of(w * R, 8), R)]
            )
            return carry

        lax.fori_loop(0, W_pad // 32, window, 0)

    return pl.kernel(
        body,
        out_type=jax.ShapeDtypeStruct((W_pad * R, 256), jnp.float32),
        mesh=mesh,
        scratch_types=[
            pltpu.VMEM((R + 8, 256), jnp.float32),
            pltpu.VMEM((R + 8, 256), jnp.float32),
            pltpu.VMEM((_K, 512), jnp.float32),
            pltpu.VMEM((_K + 16,), jnp.int32),
            pltpu.VMEM((3 * W_pad + 16,), jnp.int32),
        ],
    )


def _prep_graph(dst, e, R, W_pad, NB):
    K = _K
    perm = jnp.argsort(dst)
    sdst = jnp.take(dst, perm).astype(jnp.int32)
    wb = jnp.searchsorted(
        sdst, (jnp.arange(W_pad + 1, dtype=jnp.int32) * R).astype(sdst.dtype)
    ).astype(jnp.int32)
    lo = wb[:-1]
    hi = wb[1:]
    tstart = lo & ~7
    span = hi - tstart
    nblk = (span + K - 1) // K
    cum = jnp.cumsum(nblk).astype(jnp.int32)
    blkbase = jnp.concatenate([jnp.zeros((1,), jnp.int32), cum[:-1]])
    total = cum[-1]
    bid = jnp.arange(NB, dtype=jnp.int32)
    owner = jnp.minimum(
        jnp.searchsorted(cum, bid, side="right").astype(jnp.int32), W_pad - 1
    )
    j = bid - jnp.take(blkbase, owner)
    s = jnp.take(tstart, owner) + j * K
    slots = s[:, None] + jnp.arange(K, dtype=jnp.int32)[None, :]
    valid = (
        (slots >= jnp.take(lo, owner)[:, None])
        & (slots < jnp.take(hi, owner)[:, None])
        & (bid < total)[:, None]
    )
    g = jnp.take(sdst, jnp.clip(slots, 0, e - 1))
    relblk = jnp.where(valid, g - owner[:, None] * R, R).astype(jnp.int32)
    relblk = jnp.pad(relblk, ((0, 0), (0, K)), constant_values=R).reshape(-1)
    meta = jnp.concatenate([nblk, tstart, blkbase, jnp.zeros((16,), jnp.int32)])
    return {"perm": perm, "relblk": relblk, "meta": meta}


def _ln(x, g, b):
    mu = jnp.mean(x, axis=-1, keepdims=True)
    v = jnp.var(x, axis=-1, keepdims=True)
    return (x - mu) / jnp.sqrt(v + 1e-5) * g + b


def _mlp(p, x):
    return jax.nn.silu(_ln(x @ p["W"] + p["b"], p["g"], p["be"]))


def _rbf(r, vmin, vmax, bins):
    c = jnp.linspace(vmin, vmax, bins)
    gamma = (bins - 1) / (vmax - vmin)
    return jnp.exp(-gamma * (r[:, None] - c[None, :]) ** 2)


def _eggc(p, src, dst, n, x, y, prep, sck):
    m = (x @ p["Wsg"] + p["bsg"])[src] + (x @ p["Wdg"] + p["bdg"])[dst] \
        + y @ p["Weg"] + p["beg"]
    sig = jax.nn.sigmoid(m)
    Bh = x @ p["Wdu"] + p["bdu"]
    D = jnp.concatenate([sig * Bh[src], sig], axis=1)
    D = jnp.pad(jnp.take(D, prep["perm"], axis=0), ((0, _K), (0, 0)))
    h = sck(D, prep["relblk"], prep["meta"])[:n]
    xn = jax.nn.silu(_ln(x @ p["Wsu"] + p["bsu"] + h, p["gn"], p["bnn"]))
    yn = jax.nn.silu(_ln(m, p["ge"], p["bee"]))
    return x + xn, y + yn


def kernel(atom_feats, bond_r, angle_cos, params, edge_index, lg_edge_index):
    src, dst = edge_index[0], edge_index[1]
    lsrc, ldst = lg_edge_index[0], lg_edge_index[1]
    n = atom_feats.shape[0]
    e = bond_r.shape[0]
    tpl = angle_cos.shape[0]

    R = 128
    cw_W = 96
    cw_NB = e // _K + 2 * cw_W + 4
    lg_W = 1280
    lg_NB = tpl // _K + 2 * lg_W + 4
    sck_cw = _make_sc_segsum(e, R, cw_W, cw_NB)
    sck_lg = _make_sc_segsum(tpl, R, lg_W, lg_NB)
    prep_cw = _prep_graph(dst, e, R, cw_W, cw_NB)
    prep_lg = _prep_graph(ldst, tpl, R, lg_W, lg_NB)

    x = _mlp(params["atom_emb"], atom_feats)
    y = _mlp(params["edge_emb2"], _mlp(params["edge_emb1"], _rbf(bond_r, 0.0, 8.0, 80)))
    z = _mlp(params["angle_emb2"], _mlp(params["angle_emb1"], _rbf(angle_cos, -1.0, 1.0, 40)))
    for layer in params["alignn"]:
        x, m = _eggc(layer["node"], src, dst, n, x, y, prep_cw, sck_cw)
        y, z = _eggc(layer["edge"], lsrc, ldst, e, m, z, prep_lg, sck_lg)
    for p in params["gcn"]:
        x, y = _eggc(p, src, dst, n, x, y, prep_cw, sck_cw)
    h = jnp.mean(x, axis=0)
    out = h @ params["fc"]["W"] + params["fc"]["b"]
    return out


# speedup vs baseline: 1.3957x; 1.3957x over previous
import functools

import jax
import jax.numpy as jnp
from jax import lax
from jax.experimental import pallas as pl
from jax.experimental.pallas import tpu as pltpu
from jax.experimental.pallas import tpu_sc as plsc

_K = 32
_RS = 128


@functools.cache
def _make_sc_segsum(e, R, W_pad, NB):
    assert R % 16 == 0 and W_pad % 32 == 0
    mesh = plsc.VectorSubcoreMesh(
        core_axis_name="c", subcore_axis_name="s", num_cores=2, num_subcores=16
    )

    def body(D, relblk, meta, h, accn, accd, dbuf0, dbuf1, relbuf0, relbuf1,
             meta_v, sem0, sem1):
        c = lax.axis_index("c")
        t = lax.axis_index("s")
        wid = c * 16 + t
        pltpu.sync_copy(meta, meta_v)
        dbufs = (dbuf0, dbuf1)
        relbufs = (relbuf0, relbuf1)
        sems = (sem0, sem1)

        def window(wi, carry):
            w = wi * 32 + wid

            def zr(r, carry2):
                z = jnp.zeros((16,), jnp.float32)
                for f in range(16):
                    accn[r, pl.ds(f * 16, 16)] = z
                    accd[r, pl.ds(f * 16, 16)] = z
                return carry2

            lax.fori_loop(0, R, zr, 0)

            nb = meta_v[pl.ds(w, 16)][0]
            ts_ = meta_v[pl.ds(W_pad + w, 16)][0]
            bb = meta_v[pl.ds(2 * W_pad + w, 16)][0]

            def issue(j, b):
                s = pl.multiple_of(ts_ + j * _K, 8)
                pltpu.async_copy(D.at[pl.ds(s, _K)], dbufs[b], sems[b])
                pltpu.async_copy(
                    relblk.at[pl.ds((bb + j) * _RS, _K)],
                    relbufs[b].at[pl.ds(0, _K)],
                    sems[b],
                )

            @pl.when(nb > 0)
            def _():
                issue(0, 0)

            def process(j, b):
                pltpu.make_async_copy(D.at[pl.ds(0, _K)], dbufs[b], sems[b]).wait()
                pltpu.make_async_copy(
                    relblk.at[pl.ds(0, _K)], relbufs[b].at[pl.ds(0, _K)], sems[b]
                ).wait()

                @pl.when(j + 1 < nb)
                def _():
                    issue(j + 1, 1 - b)

                db = dbufs[b]
                rb = relbufs[b]

                def grp(g, carry3):
                    rv = rb[pl.ds(g * 4, 16)]
                    for k in range(4):
                        i = g * 4 + k
                        rel = rv[k]
                        for f in range(16):
                            sl = pl.ds(f * 16, 16)
                            accn[rel, sl] = accn[rel, sl] + db[i, sl]
                            accd[rel, sl] = accd[rel, sl] + db[i, pl.ds(256 + f * 16, 16)]
                    return carry3

                lax.fori_loop(0, _K // 4, grp, 0)

            def blk(j, carry2):
                @pl.when(j & 1 == 0)
                def _():
                    process(j, 0)

                @pl.when(j & 1 == 1)
                def _():
                    process(j, 1)

                return carry2

            lax.fori_loop(0, nb, blk, 0)

            def dv(r, carry2):
                for f in range(16):
                    sl = pl.ds(f * 16, 16)
                    accn[r, sl] = accn[r, sl] / (accd[r, sl] + 1e-6)
                return carry2

            lax.fori_loop(0, R, dv, 0)
            pltpu.sync_copy(
                accn.at[pl.ds(0, R)], h.at[pl.ds(pl.multiple_of(w * R, 8), R)]
            )
            return carry

        lax.fori_loop(0, W_pad // 32, window, 0)

    return pl.kernel(
        body,
        out_type=jax.ShapeDtypeStruct((W_pad * R, 256), jnp.float32),
        mesh=mesh,
        scratch_types=[
            pltpu.VMEM((R + 8, 256), jnp.float32),
            pltpu.VMEM((R + 8, 256), jnp.float32),
            pltpu.VMEM((_K, 512), jnp.float32),
            pltpu.VMEM((_K, 512), jnp.float32),
            pltpu.VMEM((_K + 16,), jnp.int32),
            pltpu.VMEM((_K + 16,), jnp.int32),
            pltpu.VMEM((3 * W_pad + 16,), jnp.int32),
            pltpu.SemaphoreType.DMA,
            pltpu.SemaphoreType.DMA,
        ],
    )


def _prep_graph(dst, e, R, W_pad, NB):
    K = _K
    perm = jnp.argsort(dst)
    sdst = jnp.take(dst, perm).astype(jnp.int32)
    wb = jnp.searchsorted(
        sdst, (jnp.arange(W_pad + 1, dtype=jnp.int32) * R).astype(sdst.dtype)
    ).astype(jnp.int32)
    lo = wb[:-1]
    hi = wb[1:]
    tstart = lo & ~7
    span = hi - tstart
    nblk = (span + K - 1) // K
    cum = jnp.cumsum(nblk).astype(jnp.int32)
    blkbase = jnp.concatenate([jnp.zeros((1,), jnp.int32), cum[:-1]])
    total = cum[-1]
    bid = jnp.arange(NB, dtype=jnp.int32)
    owner = jnp.minimum(
        jnp.searchsorted(cum, bid, side="right").astype(jnp.int32), W_pad - 1
    )
    j = bid - jnp.take(blkbase, owner)
    s = jnp.take(tstart, owner) + j * K
    slots = s[:, None] + jnp.arange(K, dtype=jnp.int32)[None, :]
    valid = (
        (slots >= jnp.take(lo, owner)[:, None])
        & (slots < jnp.take(hi, owner)[:, None])
        & (bid < total)[:, None]
    )
    g = jnp.take(sdst, jnp.clip(slots, 0, e - 1))
    relblk = jnp.where(valid, g - owner[:, None] * R, R).astype(jnp.int32)
    relblk = jnp.pad(relblk, ((0, 0), (0, _RS - K)), constant_values=R).reshape(-1)
    meta = jnp.concatenate([nblk, tstart, blkbase, jnp.zeros((16,), jnp.int32)])
    return {"perm": perm, "relblk": relblk, "meta": meta}


def _prep_src(prep, src):
    prep["ssrc"] = jnp.take(src, prep["perm"])
    return prep


def _ln(x, g, b):
    mu = jnp.mean(x, axis=-1, keepdims=True)
    v = jnp.var(x, axis=-1, keepdims=True)
    return (x - mu) / jnp.sqrt(v + 1e-5) * g + b


def _mlp(p, x):
    return jax.nn.silu(_ln(x @ p["W"] + p["b"], p["g"], p["be"]))


def _rbf(r, vmin, vmax, bins):
    c = jnp.linspace(vmin, vmax, bins)
    gamma = (bins - 1) / (vmax - vmin)
    return jnp.exp(-gamma * (r[:, None] - c[None, :]) ** 2)


def _eggc(p, src, dst, n, x, y, prep, sck):
    m = (x @ p["Wsg"] + p["bsg"])[src] + (x @ p["Wdg"] + p["bdg"])[dst] \
        + y @ p["Weg"] + p["beg"]
    Bh = x @ p["Wdu"] + p["bdu"]
    sig_s = jax.nn.sigmoid(jnp.take(m, prep["perm"], axis=0))
    D = jnp.concatenate([sig_s * jnp.take(Bh, prep["ssrc"], axis=0), sig_s], axis=1)
    D = jnp.pad(D, ((0, _K), (0, 0)))
    h = sck(D, prep["relblk"], prep["meta"])[:n]
    xn = jax.nn.silu(_ln(x @ p["Wsu"] + p["bsu"] + h, p["gn"], p["bnn"]))
    yn = jax.nn.silu(_ln(m, p["ge"], p["bee"]))
    return x + xn, y + yn


def kernel(atom_feats, bond_r, angle_cos, params, edge_index, lg_edge_index):
    src, dst = edge_index[0], edge_index[1]
    lsrc, ldst = lg_edge_index[0], lg_edge_index[1]
    n = atom_feats.shape[0]
    e = bond_r.shape[0]
    tpl = angle_cos.shape[0]

    R = 128
    cw_W = 96
    cw_NB = e // _K + 2 * cw_W + 4
    lg_W = 1280
    lg_NB = tpl // _K + 2 * lg_W + 4
    sck_cw = _make_sc_segsum(e, R, cw_W, cw_NB)
    sck_lg = _make_sc_segsum(tpl, R, lg_W, lg_NB)
    prep_cw = _prep_src(_prep_graph(dst, e, R, cw_W, cw_NB), src)
    prep_lg = _prep_src(_prep_graph(ldst, tpl, R, lg_W, lg_NB), lsrc)

    x = _mlp(params["atom_emb"], atom_feats)
    y = _mlp(params["edge_emb2"], _mlp(params["edge_emb1"], _rbf(bond_r, 0.0, 8.0, 80)))
    z = _mlp(params["angle_emb2"], _mlp(params["angle_emb1"], _rbf(angle_cos, -1.0, 1.0, 40)))
    for layer in params["alignn"]:
        x, m = _eggc(layer["node"], src, dst, n, x, y, prep_cw, sck_cw)
        y, z = _eggc(layer["edge"], lsrc, ldst, e, m, z, prep_lg, sck_lg)
    for p in params["gcn"]:
        x, y = _eggc(p, src, dst, n, x, y, prep_cw, sck_cw)
    h = jnp.mean(x, axis=0)
    out = h @ params["fc"]["W"] + params["fc"]["b"]
    return out


# speedup vs baseline: 1.4070x; 1.0081x over previous
import functools

import jax
import jax.numpy as jnp
from jax import lax
from jax.experimental import pallas as pl
from jax.experimental.pallas import tpu as pltpu
from jax.experimental.pallas import tpu_sc as plsc

_K = 32
_RS = 128


@functools.cache
def _make_sc_segsum(e, R, W_pad, NB):
    assert R % 16 == 0 and W_pad % 32 == 0
    mesh = plsc.VectorSubcoreMesh(
        core_axis_name="c", subcore_axis_name="s", num_cores=2, num_subcores=16
    )

    def body(D, relblk, meta, h, accn, accd, dbuf0, dbuf1, relbuf0, relbuf1,
             meta_v, sem0, sem1):
        c = lax.axis_index("c")
        t = lax.axis_index("s")
        wid = c * 16 + t
        pltpu.sync_copy(meta, meta_v)
        dbufs = (dbuf0, dbuf1)
        relbufs = (relbuf0, relbuf1)
        sems = (sem0, sem1)

        def window(wi, carry):
            w = wi * 32 + wid

            def zr(r, carry2):
                z = jnp.zeros((16,), jnp.float32)
                for f in range(16):
                    accn[r, pl.ds(f * 16, 16)] = z
                    accd[r, pl.ds(f * 16, 16)] = z
                return carry2

            lax.fori_loop(0, R, zr, 0)

            nb = meta_v[pl.ds(w, 16)][0]
            ts_ = meta_v[pl.ds(W_pad + w, 16)][0]
            bb = meta_v[pl.ds(2 * W_pad + w, 16)][0]

            def issue(j, b):
                s = pl.multiple_of(ts_ + j * _K, 8)
                pltpu.async_copy(D.at[pl.ds(s, _K)], dbufs[b], sems[b])
                pltpu.async_copy(
                    relblk.at[pl.ds((bb + j) * _RS, _K)],
                    relbufs[b].at[pl.ds(0, _K)],
                    sems[b],
                )

            @pl.when(nb > 0)
            def _():
                issue(0, 0)

            def process(j, b):
                pltpu.make_async_copy(D.at[pl.ds(0, _K)], dbufs[b], sems[b]).wait()
                pltpu.make_async_copy(
                    relblk.at[pl.ds(0, _K)], relbufs[b].at[pl.ds(0, _K)], sems[b]
                ).wait()

                @pl.when(j + 1 < nb)
                def _():
                    issue(j + 1, 1 - b)

                db = dbufs[b]
                rb = relbufs[b]

                def grp(g, carry3):
                    rv = rb[pl.ds(g * 4, 16)]
                    for k in range(4):
                        i = g * 4 + k
                        rel = rv[k]
                        for f in range(16):
                            sl = pl.ds(f * 16, 16)
                            accn[rel, sl] = accn[rel, sl] + db[i, sl]
                            accd[rel, sl] = accd[rel, sl] + db[i, pl.ds(256 + f * 16, 16)]
                    return carry3

                lax.fori_loop(0, _K // 4, grp, 0)

            def blk(j, carry2):
                @pl.when(j & 1 == 0)
                def _():
                    process(j, 0)

                @pl.when(j & 1 == 1)
                def _():
                    process(j, 1)

                return carry2

            lax.fori_loop(0, nb, blk, 0)

            def dv(r, carry2):
                for f in range(16):
                    sl = pl.ds(f * 16, 16)
                    accn[r, sl] = accn[r, sl] / (accd[r, sl] + 1e-6)
                return carry2

            lax.fori_loop(0, R, dv, 0)
            pltpu.sync_copy(
                accn.at[pl.ds(0, R)], h.at[pl.ds(pl.multiple_of(w * R, 8), R)]
            )
            return carry

        lax.fori_loop(0, W_pad // 32, window, 0)

    return pl.kernel(
        body,
        out_type=jax.ShapeDtypeStruct((W_pad * R, 256), jnp.float32),
        mesh=mesh,
        scratch_types=[
            pltpu.VMEM((R + 8, 256), jnp.float32),
            pltpu.VMEM((R + 8, 256), jnp.float32),
            pltpu.VMEM((_K, 512), jnp.float32),
            pltpu.VMEM((_K, 512), jnp.float32),
            pltpu.VMEM((_K + 16,), jnp.int32),
            pltpu.VMEM((_K + 16,), jnp.int32),
            pltpu.VMEM((3 * W_pad + 16,), jnp.int32),
            pltpu.SemaphoreType.DMA,
            pltpu.SemaphoreType.DMA,
        ],
    )


def _prep_graph(dst, e, R, W_pad, NB):
    K = _K
    perm = jnp.argsort(dst)
    sdst = jnp.take(dst, perm).astype(jnp.int32)
    wb = jnp.searchsorted(
        sdst, (jnp.arange(W_pad + 1, dtype=jnp.int32) * R).astype(sdst.dtype)
    ).astype(jnp.int32)
    lo = wb[:-1]
    hi = wb[1:]
    tstart = lo & ~7
    span = hi - tstart
    nblk = (span + K - 1) // K
    cum = jnp.cumsum(nblk).astype(jnp.int32)
    blkbase = jnp.concatenate([jnp.zeros((1,), jnp.int32), cum[:-1]])
    total = cum[-1]
    bid = jnp.arange(NB, dtype=jnp.int32)
    owner = jnp.minimum(
        jnp.searchsorted(cum, bid, side="right").astype(jnp.int32), W_pad - 1
    )
    j = bid - jnp.take(blkbase, owner)
    s = jnp.take(tstart, owner) + j * K
    slots = s[:, None] + jnp.arange(K, dtype=jnp.int32)[None, :]
    valid = (
        (slots >= jnp.take(lo, owner)[:, None])
        & (slots < jnp.take(hi, owner)[:, None])
        & (bid < total)[:, None]
    )
    g = jnp.take(sdst, jnp.clip(slots, 0, e - 1))
    relblk = jnp.where(valid, g - owner[:, None] * R, R).astype(jnp.int32)
    relblk = jnp.pad(relblk, ((0, 0), (0, _RS - K)), constant_values=R).reshape(-1)
    meta = jnp.concatenate([nblk, tstart, blkbase, jnp.zeros((16,), jnp.int32)])
    return {"perm": perm, "relblk": relblk, "meta": meta}


def _prep_src(prep, src):
    prep["ssrc"] = jnp.take(src, prep["perm"])
    return prep


def _ln(x, g, b):
    mu = jnp.mean(x, axis=-1, keepdims=True)
    v = jnp.var(x, axis=-1, keepdims=True)
    return (x - mu) / jnp.sqrt(v + 1e-5) * g + b


def _mlp(p, x):
    return jax.nn.silu(_ln(x @ p["W"] + p["b"], p["g"], p["be"]))


def _rbf(r, vmin, vmax, bins):
    c = jnp.linspace(vmin, vmax, bins)
    gamma = (bins - 1) / (vmax - vmin)
    return jnp.exp(-gamma * (r[:, None] - c[None, :]) ** 2)


def _bmm(a, w):
    return jax.lax.dot(
        a.astype(jnp.bfloat16), w.astype(jnp.bfloat16),
        preferred_element_type=jnp.float32,
    )


def _eggc(p, src, dst, n, x, y, prep, sck):
    m = (_bmm(x, p["Wsg"]) + p["bsg"])[src] + (_bmm(x, p["Wdg"]) + p["bdg"])[dst] \
        + _bmm(y, p["Weg"]) + p["beg"]
    Bh = _bmm(x, p["Wdu"]) + p["bdu"]
    sig_s = jax.nn.sigmoid(jnp.take(m, prep["perm"], axis=0))
    D = jnp.concatenate([sig_s * jnp.take(Bh, prep["ssrc"], axis=0), sig_s], axis=1)
    D = jnp.pad(D, ((0, _K), (0, 0)))
    h = sck(D, prep["relblk"], prep["meta"])[:n]
    xn = jax.nn.silu(_ln(_bmm(x, p["Wsu"]) + p["bsu"] + h, p["gn"], p["bnn"]))
    yn = jax.nn.silu(_ln(m, p["ge"], p["bee"]))
    return x + xn, y + yn


def kernel(atom_feats, bond_r, angle_cos, params, edge_index, lg_edge_index):
    src, dst = edge_index[0], edge_index[1]
    lsrc, ldst = lg_edge_index[0], lg_edge_index[1]
    n = atom_feats.shape[0]
    e = bond_r.shape[0]
    tpl = angle_cos.shape[0]

    R = 128
    cw_W = 96
    cw_NB = e // _K + 2 * cw_W + 4
    lg_W = 1280
    lg_NB = tpl // _K + 2 * lg_W + 4
    sck_cw = _make_sc_segsum(e, R, cw_W, cw_NB)
    sck_lg = _make_sc_segsum(tpl, R, lg_W, lg_NB)
    prep_cw = _prep_src(_prep_graph(dst, e, R, cw_W, cw_NB), src)
    prep_lg = _prep_src(_prep_graph(ldst, tpl, R, lg_W, lg_NB), lsrc)

    x = _mlp(params["atom_emb"], atom_feats)
    y = _mlp(params["edge_emb2"], _mlp(params["edge_emb1"], _rbf(bond_r, 0.0, 8.0, 80)))
    z = _mlp(params["angle_emb2"], _mlp(params["angle_emb1"], _rbf(angle_cos, -1.0, 1.0, 40)))
    for layer in params["alignn"]:
        x, m = _eggc(layer["node"], src, dst, n, x, y, prep_cw, sck_cw)
        y, z = _eggc(layer["edge"], lsrc, ldst, e, m, z, prep_lg, sck_lg)
    for p in params["gcn"]:
        x, y = _eggc(p, src, dst, n, x, y, prep_cw, sck_cw)
    h = jnp.mean(x, axis=0)
    out = h @ params["fc"]["W"] + params["fc"]["b"]
    return out


# speedup vs baseline: 1.5161x; 1.0776x over previous
import functools

import jax
import jax.numpy as jnp
from jax import lax
from jax.experimental import pallas as pl
from jax.experimental.pallas import tpu as pltpu
from jax.experimental.pallas import tpu_sc as plsc

_K = 32
_RS = 128


@functools.cache
def _make_sc_segsum(e, R, W_pad, NB):
    assert R % 16 == 0 and W_pad % 32 == 0
    mesh = plsc.VectorSubcoreMesh(
        core_axis_name="c", subcore_axis_name="s", num_cores=2, num_subcores=16
    )

    def body(D, relblk, meta, h, accn, accd, dbuf0, dbuf1, relbuf0, relbuf1,
             meta_v, sem0, sem1):
        c = lax.axis_index("c")
        t = lax.axis_index("s")
        wid = c * 16 + t
        pltpu.sync_copy(meta, meta_v)
        dbufs = (dbuf0, dbuf1)
        relbufs = (relbuf0, relbuf1)
        sems = (sem0, sem1)

        def window(wi, carry):
            w = wi * 32 + wid

            def zr(r, carry2):
                z = jnp.zeros((16,), jnp.float32)
                for f in range(16):
                    accn[r, pl.ds(f * 16, 16)] = z
                    accd[r, pl.ds(f * 16, 16)] = z
                return carry2

            lax.fori_loop(0, R, zr, 0)

            nb = meta_v[pl.ds(w, 16)][0]
            ts_ = meta_v[pl.ds(W_pad + w, 16)][0]
            bb = meta_v[pl.ds(2 * W_pad + w, 16)][0]

            def issue(j, b):
                s = pl.multiple_of(ts_ + j * _K, 8)
                pltpu.async_copy(D.at[pl.ds(s, _K)], dbufs[b], sems[b])
                pltpu.async_copy(
                    relblk.at[pl.ds((bb + j) * _RS, _K)],
                    relbufs[b].at[pl.ds(0, _K)],
                    sems[b],
                )

            @pl.when(nb > 0)
            def _():
                issue(0, 0)

            def process(j, b):
                pltpu.make_async_copy(D.at[pl.ds(0, _K)], dbufs[b], sems[b]).wait()
                pltpu.make_async_copy(
                    relblk.at[pl.ds(0, _K)], relbufs[b].at[pl.ds(0, _K)], sems[b]
                ).wait()

                @pl.when(j + 1 < nb)
                def _():
                    issue(j + 1, 1 - b)

                db = dbufs[b]
                rb = relbufs[b]

                def grp(g, carry3):
                    rv = rb[pl.ds(g * 4, 16)]
                    for k in range(4):
                        i = g * 4 + k
                        rel = rv[k]
                        for f in range(16):
                            sl = pl.ds(f * 16, 16)
                            plsc.addupdate(accn.at[rel, sl], db[i, sl])
                            plsc.addupdate(
                                accd.at[rel, sl], db[i, pl.ds(256 + f * 16, 16)]
                            )
                    return carry3

                lax.fori_loop(0, _K // 4, grp, 0)

            def blk(j, carry2):
                @pl.when(j & 1 == 0)
                def _():
                    process(j, 0)

                @pl.when(j & 1 == 1)
                def _():
                    process(j, 1)

                return carry2

            lax.fori_loop(0, nb, blk, 0)

            def dv(r, carry2):
                for f in range(16):
                    sl = pl.ds(f * 16, 16)
                    accn[r, sl] = accn[r, sl] / (accd[r, sl] + 1e-6)
                return carry2

            lax.fori_loop(0, R, dv, 0)
            pltpu.sync_copy(
                accn.at[pl.ds(0, R)], h.at[pl.ds(pl.multiple_of(w * R, 8), R)]
            )
            return carry

        lax.fori_loop(0, W_pad // 32, window, 0)

    return pl.kernel(
        body,
        out_type=jax.ShapeDtypeStruct((W_pad * R, 256), jnp.float32),
        mesh=mesh,
        scratch_types=[
            pltpu.VMEM((R + 8, 256), jnp.float32),
            pltpu.VMEM((R + 8, 256), jnp.float32),
            pltpu.VMEM((_K, 512), jnp.float32),
            pltpu.VMEM((_K, 512), jnp.float32),
            pltpu.VMEM((_K + 16,), jnp.int32),
            pltpu.VMEM((_K + 16,), jnp.int32),
            pltpu.VMEM((3 * W_pad + 16,), jnp.int32),
            pltpu.SemaphoreType.DMA,
            pltpu.SemaphoreType.DMA,
        ],
    )


def _prep_graph(dst, e, R, W_pad, NB):
    K = _K
    perm = jnp.argsort(dst)
    sdst = jnp.take(dst, perm).astype(jnp.int32)
    wb = jnp.searchsorted(
        sdst, (jnp.arange(W_pad + 1, dtype=jnp.int32) * R).astype(sdst.dtype)
    ).astype(jnp.int32)
    lo = wb[:-1]
    hi = wb[1:]
    tstart = lo & ~7
    span = hi - tstart
    nblk = (span + K - 1) // K
    cum = jnp.cumsum(nblk).astype(jnp.int32)
    blkbase = jnp.concatenate([jnp.zeros((1,), jnp.int32), cum[:-1]])
    total = cum[-1]
    bid = jnp.arange(NB, dtype=jnp.int32)
    owner = jnp.minimum(
        jnp.searchsorted(cum, bid, side="right").astype(jnp.int32), W_pad - 1
    )
    j = bid - jnp.take(blkbase, owner)
    s = jnp.take(tstart, owner) + j * K
    slots = s[:, None] + jnp.arange(K, dtype=jnp.int32)[None, :]
    valid = (
        (slots >= jnp.take(lo, owner)[:, None])
        & (slots < jnp.take(hi, owner)[:, None])
        & (bid < total)[:, None]
    )
    g = jnp.take(sdst, jnp.clip(slots, 0, e - 1))
    relblk = jnp.where(valid, g - owner[:, None] * R, R).astype(jnp.int32)
    relblk = jnp.pad(relblk, ((0, 0), (0, _RS - K)), constant_values=R).reshape(-1)
    meta = jnp.concatenate([nblk, tstart, blkbase, jnp.zeros((16,), jnp.int32)])
    return {"perm": perm, "relblk": relblk, "meta": meta}


def _prep_src(prep, src):
    prep["ssrc"] = jnp.take(src, prep["perm"])
    return prep


def _ln(x, g, b):
    mu = jnp.mean(x, axis=-1, keepdims=True)
    v = jnp.var(x, axis=-1, keepdims=True)
    return (x - mu) / jnp.sqrt(v + 1e-5) * g + b


def _mlp(p, x):
    return jax.nn.silu(_ln(x @ p["W"] + p["b"], p["g"], p["be"]))


def _rbf(r, vmin, vmax, bins):
    c = jnp.linspace(vmin, vmax, bins)
    gamma = (bins - 1) / (vmax - vmin)
    return jnp.exp(-gamma * (r[:, None] - c[None, :]) ** 2)


def _bmm(a, w):
    return jax.lax.dot(
        a.astype(jnp.bfloat16), w.astype(jnp.bfloat16),
        preferred_element_type=jnp.float32,
    )


def _eggc(p, src, dst, n, x, y, prep, sck):
    m = (_bmm(x, p["Wsg"]) + p["bsg"])[src] + (_bmm(x, p["Wdg"]) + p["bdg"])[dst] \
        + _bmm(y, p["Weg"]) + p["beg"]
    Bh = _bmm(x, p["Wdu"]) + p["bdu"]
    sig_s = jax.nn.sigmoid(jnp.take(m, prep["perm"], axis=0))
    D = jnp.concatenate([sig_s * jnp.take(Bh, prep["ssrc"], axis=0), sig_s], axis=1)
    D = jnp.pad(D, ((0, _K), (0, 0)))
    h = sck(D, prep["relblk"], prep["meta"])[:n]
    xn = jax.nn.silu(_ln(_bmm(x, p["Wsu"]) + p["bsu"] + h, p["gn"], p["bnn"]))
    yn = jax.nn.silu(_ln(m, p["ge"], p["bee"]))
    return x + xn, y + yn


def kernel(atom_feats, bond_r, angle_cos, params, edge_index, lg_edge_index):
    src, dst = edge_index[0], edge_index[1]
    lsrc, ldst = lg_edge_index[0], lg_edge_index[1]
    n = atom_feats.shape[0]
    e = bond_r.shape[0]
    tpl = angle_cos.shape[0]

    R = 128
    cw_W = 96
    cw_NB = e // _K + 2 * cw_W + 4
    lg_W = 1280
    lg_NB = tpl // _K + 2 * lg_W + 4
    sck_cw = _make_sc_segsum(e, R, cw_W, cw_NB)
    sck_lg = _make_sc_segsum(tpl, R, lg_W, lg_NB)
    prep_cw = _prep_src(_prep_graph(dst, e, R, cw_W, cw_NB), src)
    prep_lg = _prep_src(_prep_graph(ldst, tpl, R, lg_W, lg_NB), lsrc)

    x = _mlp(params["atom_emb"], atom_feats)
    y = _mlp(params["edge_emb2"], _mlp(params["edge_emb1"], _rbf(bond_r, 0.0, 8.0, 80)))
    z = _mlp(params["angle_emb2"], _mlp(params["angle_emb1"], _rbf(angle_cos, -1.0, 1.0, 40)))
    for layer in params["alignn"]:
        x, m = _eggc(layer["node"], src, dst, n, x, y, prep_cw, sck_cw)
        y, z = _eggc(layer["edge"], lsrc, ldst, e, m, z, prep_lg, sck_lg)
    for p in params["gcn"]:
        x, y = _eggc(p, src, dst, n, x, y, prep_cw, sck_cw)
    h = jnp.mean(x, axis=0)
    out = h @ params["fc"]["W"] + params["fc"]["b"]
    return out
